# Initial kernel scaffold; baseline (speedup 1.0000x reference)
import jax
import jax.numpy as jnp
from jax.experimental import pallas as pl


def kernel(x, index):
    raise NotImplementedError("write your pallas kernel here")


# speedup vs baseline: 4.8192x; 4.8192x over previous
import functools

import jax
import jax.numpy as jnp
import numpy as np
from jax import lax
from jax.experimental import pallas as pl
from jax.experimental.pallas import tpu as pltpu
from jax.experimental.pallas import tpu_sc as plsc

E = 320000
D = 128
N = 10000
NC, NS = 2, 16
HALF = N // NC
TBL = HALF + 8
CH = 128
NCHUNK = E // CH
CW = 16
WPT = 312

_mesh = plsc.VectorSubcoreMesh(core_axis_name="c", subcore_axis_name="s")


@functools.partial(
    pl.kernel,
    out_type=(
        jax.ShapeDtypeStruct((N, D), jnp.float32),
        jax.ShapeDtypeStruct((N, CW), jnp.float32),
    ),
    mesh=_mesh,
    scratch_types=[
        pltpu.VMEM((CH,), jnp.int32),
        pltpu.VMEM((CH,), jnp.int32),
        pltpu.VMEM((CH, D), jnp.float32),
        pltpu.VMEM((CH + 16, CW), jnp.float32),
        pltpu.VMEM_SHARED((TBL, D), jnp.float32),
        pltpu.VMEM_SHARED((TBL, CW), jnp.float32),
    ],
)
def _scatter_halves(x_hbm, idx_hbm, aux_hbm, sums_out, cnts_out,
                    idx_v, idx2_v, x_v, aux_v, sums_sh, cnts_sh):
    c = lax.axis_index("c")
    s = lax.axis_index("s")

    zero16 = jnp.zeros((16,), jnp.float32)
    one16 = jnp.ones((16,), jnp.float32)

    def initrow(i, carry):
        for j in range(D // 16):
            x_v[i, pl.ds(16 * j, 16)] = zero16
        return carry

    lax.fori_loop(0, CH, initrow, 0)
    pltpu.sync_copy(aux_hbm, aux_v)

    nclr = jnp.where(s == NS - 1, 13, 20)

    def clr(i, carry):
        row = s * 320 + i * 16
        pltpu.sync_copy(x_v.at[pl.ds(0, 16)], sums_sh.at[pl.ds(row, 16)])
        pltpu.sync_copy(aux_v.at[pl.ds(CH, 16)], cnts_sh.at[pl.ds(row, 16)])
        return carry

    lax.fori_loop(0, nclr, clr, 0)
    plsc.subcore_barrier()

    lo = c * HALF
    nch = jnp.where(s < NCHUNK - 156 * NS, 157, 156)

    def chunk(t, carry):
        off = (t * NS + s) * CH
        pltpu.sync_copy(idx_hbm.at[pl.ds(off, CH)], idx_v)
        first = idx_v[pl.ds(0, 16)][0]
        last = idx_v[pl.ds(CH - 16, 16)][15]

        @pl.when((last >= lo) & (first < lo + HALF))
        def _():
            pltpu.sync_copy(x_hbm.at[pl.ds(off, CH)], x_v)
            for j in range(CH // 16):
                v = idx_v[pl.ds(16 * j, 16)] - lo
                ok = (v >= 0) & (v < HALF)
                idx2_v[pl.ds(16 * j, 16)] = jnp.where(ok, v, HALF)
            pltpu.sync_copy(x_v, sums_sh.at[idx2_v], add=True)
            pltpu.sync_copy(aux_v.at[pl.ds(0, CH)], cnts_sh.at[idx2_v], add=True)

        return carry

    lax.fori_loop(0, nch, chunk, 0)
    plsc.subcore_barrier()

    row0 = s * WPT
    pltpu.sync_copy(sums_sh.at[pl.ds(row0, WPT)],
                    sums_out.at[pl.ds(lo + row0, WPT)])
    pltpu.sync_copy(cnts_sh.at[pl.ds(row0, WPT)],
                    cnts_out.at[pl.ds(lo + row0, WPT)])

    @pl.when(s == NS - 1)
    def _():
        pltpu.sync_copy(sums_sh.at[pl.ds(NS * WPT, 8)],
                        sums_out.at[pl.ds(lo + NS * WPT, 8)])
        pltpu.sync_copy(cnts_sh.at[pl.ds(NS * WPT, 8)],
                        cnts_out.at[pl.ds(lo + NS * WPT, 8)])


def _combine_body(s_ref, c_ref, o_ref):
    cnt = c_ref[:, 0:1]
    o_ref[...] = s_ref[...] / jnp.maximum(cnt, 1.0)


_BN = 1000


_AUX = np.concatenate([np.ones((CH, CW), np.float32),
                       np.zeros((16, CW), np.float32)])


def kernel(x, index):
    sums, cnts = _scatter_halves(x, index, jnp.asarray(_AUX))
    return pl.pallas_call(
        _combine_body,
        grid=(N // _BN,),
        in_specs=[
            pl.BlockSpec((_BN, D), lambda i: (i, 0)),
            pl.BlockSpec((_BN, CW), lambda i: (i, 0)),
        ],
        out_specs=pl.BlockSpec((_BN, D), lambda i: (i, 0)),
        out_shape=jax.ShapeDtypeStruct((N, D), jnp.float32),
    )(sums, cnts)
